# baseline (speedup 1.0000x reference)
import jax
import jax.numpy as jnp
from jax.experimental import pallas as pl
from jax.experimental.pallas import tpu as pltpu

_NUM_ANCHORS = 3
_NUM_CLASSES = 80
_ATTRS = 5 + _NUM_CLASSES
_H = 52
_W = 52
_S = _H * _W
_STRIDE = 8.0
_ANCHOR_W = (10.0, 16.0, 33.0)
_ANCHOR_H = (13.0, 30.0, 23.0)


def _decode_kernel(in_ref, out_ref):
    a = pl.program_id(1)
    col = jax.lax.broadcasted_iota(jnp.int32, (1, _S), 1)
    gx = (col % _W).astype(jnp.float32)
    gy = (col // _W).astype(jnp.float32)
    aw = jnp.where(a == 0, _ANCHOR_W[0], jnp.where(a == 1, _ANCHOR_W[1], _ANCHOR_W[2]))
    ah = jnp.where(a == 0, _ANCHOR_H[0], jnp.where(a == 1, _ANCHOR_H[1], _ANCHOR_H[2]))

    blk = in_ref[0, 0]
    sig = jax.nn.sigmoid(blk)
    bx = (sig[0:1] + gx) * _STRIDE
    by = (sig[1:2] + gy) * _STRIDE
    bw = jnp.exp(blk[2:3]) * aw
    bh = jnp.exp(blk[3:4]) * ah
    dec = jnp.concatenate([bx, by, bw, bh, sig[4:]], axis=0)
    out_ref[0] = dec.T


def kernel(input):
    bs = input.shape[0]
    flat = input.reshape(bs, _NUM_ANCHORS, _ATTRS, _S)
    out = pl.pallas_call(
        _decode_kernel,
        grid=(bs, _NUM_ANCHORS),
        in_specs=[pl.BlockSpec((1, 1, _ATTRS, _S), lambda b, a: (b, a, 0, 0))],
        out_specs=pl.BlockSpec((1, _S, _ATTRS), lambda b, a: (b, a, 0)),
        out_shape=jax.ShapeDtypeStruct((bs, _NUM_ANCHORS * _S, _ATTRS), jnp.float32),
        compiler_params=pltpu.CompilerParams(
            dimension_semantics=("parallel", "parallel"),
        ),
    )(flat)
    return out


# speedup vs baseline: 2.0609x; 2.0609x over previous
import jax
import jax.numpy as jnp
from jax.experimental import pallas as pl

_NUM_ANCHORS = 3
_NUM_CLASSES = 80
_ATTRS = 5 + _NUM_CLASSES
_H = 52
_W = 52
_S = _H * _W
_STRIDE = 8.0
_ANCHOR_W = (10.0, 16.0, 33.0)
_ANCHOR_H = (13.0, 30.0, 23.0)


def _sigmoid(x):
    return 0.5 * jnp.tanh(0.5 * x) + 0.5


def _decode_kernel(in_ref, out_ref):
    col = jax.lax.broadcasted_iota(jnp.int32, (1, _S), 1)
    gx = (col % _W).astype(jnp.float32)
    gy = (col // _W).astype(jnp.float32)

    for a in range(_NUM_ANCHORS):
        blk = in_ref[0, a * _ATTRS:(a + 1) * _ATTRS, :]
        sig = _sigmoid(blk)
        bx = (sig[0:1] + gx) * _STRIDE
        by = (sig[1:2] + gy) * _STRIDE
        bw = jnp.exp(blk[2:3]) * _ANCHOR_W[a]
        bh = jnp.exp(blk[3:4]) * _ANCHOR_H[a]
        dec = jnp.concatenate([bx, by, bw, bh, sig[4:]], axis=0)
        out_ref[0, a * _S:(a + 1) * _S, :] = dec.T


def kernel(input):
    bs = input.shape[0]
    flat = input.reshape(bs, _NUM_ANCHORS * _ATTRS, _S)
    out = pl.pallas_call(
        _decode_kernel,
        grid=(bs,),
        in_specs=[pl.BlockSpec((1, _NUM_ANCHORS * _ATTRS, _S), lambda b: (b, 0, 0))],
        out_specs=pl.BlockSpec((1, _NUM_ANCHORS * _S, _ATTRS), lambda b: (b, 0, 0)),
        out_shape=jax.ShapeDtypeStruct((bs, _NUM_ANCHORS * _S, _ATTRS), jnp.float32),
    )(flat)
    return out
